# baseline (speedup 1.0000x reference)
import jax
import jax.numpy as jnp
from jax.experimental import pallas as pl
from jax.experimental.pallas import tpu as pltpu

_N = 10000
_D = 128
_BM1 = 400
_BM2 = 1000
_F8 = jnp.float8_e4m3fn
_FMAX = 224.0


def _l1_body(x_ref, w1_ref, adj_ref, b_ref, w_ref, q_ref, o_ref, s_ref):
    @pl.when(pl.program_id(0) == 0)
    def _():
        s_ref[...] = jnp.dot(x_ref[...].astype(jnp.bfloat16),
                             w1_ref[...].astype(jnp.bfloat16),
                             preferred_element_type=jnp.float32
                             ).astype(jnp.bfloat16)
    a = adj_ref[...]
    q_ref[...] = a.astype(_F8)
    h = jnp.maximum(
        jnp.dot(a.astype(jnp.bfloat16), s_ref[...],
                preferred_element_type=jnp.float32) + b_ref[...], 0.0)
    o_ref[...] = jnp.dot(h.astype(jnp.bfloat16), w_ref[...],
                         preferred_element_type=jnp.float32
                         ).astype(jnp.bfloat16)


def _quantize(s, sq_ref, invc_ref, mu_ref):
    s = s.astype(jnp.float32)
    m = jnp.sum(s, axis=0, keepdims=True) * (1.0 / _N)
    r = s - m
    cmax = jnp.maximum(jnp.max(jnp.abs(r), axis=0, keepdims=True), 1e-30)
    sq_ref[:, :_D] = (r * (_FMAX / cmax)).astype(_F8)
    invc_ref[...] = cmax * (1.0 / _FMAX)
    mu_ref[...] = m


def _l23_body(q_ref, s_ref, b2_ref, b3_ref, w3_ref, o_ref,
              sq_ref, invc_ref, mu_ref, s3_ref):
    i = pl.program_id(0)
    nb = _N // _BM2

    @pl.when(i == 0)
    def _():
        sq_ref[:, _D:] = jnp.ones((_N, _D), _F8)
        _quantize(s_ref[...], sq_ref, invc_ref, mu_ref)

    @pl.when(i == nb)
    def _():
        _quantize(s3_ref[...], sq_ref, invc_ref, mu_ref)

    acc = jnp.dot(q_ref[...], sq_ref[...],
                  preferred_element_type=jnp.float32)
    val = acc[:, :_D] * invc_ref[...] + acc[:, _D:] * mu_ref[...]

    @pl.when(i < nb)
    def _():
        h = jnp.maximum(val + b2_ref[...], 0.0)
        s3_ref[pl.ds(i * _BM2, _BM2), :] = jnp.dot(
            h.astype(jnp.bfloat16), w3_ref[...].astype(jnp.bfloat16),
            preferred_element_type=jnp.float32).astype(jnp.bfloat16)

    @pl.when(i >= nb)
    def _():
        o_ref[...] = val + b3_ref[...]


def _layer1(x, w1, adj, b, w):
    return pl.pallas_call(
        _l1_body,
        grid=(_N // _BM1,),
        in_specs=[
            pl.BlockSpec((_N, _D), lambda i: (0, 0)),
            pl.BlockSpec((_D, _D), lambda i: (0, 0)),
            pl.BlockSpec((_BM1, _N), lambda i: (i, 0)),
            pl.BlockSpec((1, _D), lambda i: (0, 0)),
            pl.BlockSpec((_D, _D), lambda i: (0, 0)),
        ],
        out_specs=[
            pl.BlockSpec((_BM1, _N), lambda i: (i, 0)),
            pl.BlockSpec((_BM1, _D), lambda i: (i, 0)),
        ],
        out_shape=[
            jax.ShapeDtypeStruct((_N, _N), _F8),
            jax.ShapeDtypeStruct((_N, _D), jnp.bfloat16),
        ],
        scratch_shapes=[pltpu.VMEM((_N, _D), jnp.bfloat16)],
    )(x, w1, adj, b, w)


def _layer23(q, s, b2, b3, w3):
    nb = _N // _BM2
    return pl.pallas_call(
        _l23_body,
        grid=(2 * nb,),
        in_specs=[
            pl.BlockSpec((_BM2, _N), lambda i, nb=nb: (i % nb, 0)),
            pl.BlockSpec((_N, _D), lambda i: (0, 0)),
            pl.BlockSpec((1, _D), lambda i: (0, 0)),
            pl.BlockSpec((1, _D), lambda i: (0, 0)),
            pl.BlockSpec((_D, _D), lambda i: (0, 0)),
        ],
        out_specs=pl.BlockSpec((_BM2, _D),
                               lambda i, nb=nb: (jnp.maximum(i - nb, 0), 0)),
        out_shape=jax.ShapeDtypeStruct((_N, _D), jnp.float32),
        scratch_shapes=[pltpu.VMEM((_N, 2 * _D), _F8),
                        pltpu.VMEM((1, _D), jnp.float32),
                        pltpu.VMEM((1, _D), jnp.float32),
                        pltpu.VMEM((_N, _D), jnp.bfloat16)],
    )(q, s, b2, b3, w3)


def kernel(x, adj, W1, b1, W2, b2, W3, b3):
    b1r = b1.reshape(1, _D)
    b2r = b2.reshape(1, _D)
    b3r = b3.reshape(1, _D)
    q, s2 = _layer1(x, W1, adj, b1r, W2.astype(jnp.bfloat16))
    return _layer23(q, s2, b2r, b3r, W3)
